# Initial kernel scaffold; baseline (speedup 1.0000x reference)
import jax
import jax.numpy as jnp
from jax.experimental import pallas as pl


def kernel(img1, img2, feat_w, feat_b, reg_w, reg_b):
    raise NotImplementedError("write your pallas kernel here")


# speedup vs baseline: 3.3403x; 3.3403x over previous
import jax
import jax.numpy as jnp
from jax.experimental import pallas as pl
from jax.experimental.pallas import tpu as pltpu

_H = 15
_HW = _H * _H
_HWP = 256
_EPS = 1e-6


def _fused_kernel(x1_ref, x2_ref, fw_ref, fb_ref, w1_ref, w2_ref, b_ref,
                  o1_ref, o2_ref):
    x1 = x1_ref[0]
    x2 = x2_ref[0]
    fw = fw_ref[...]
    fb = fb_ref[...]

    lane = jax.lax.broadcasted_iota(jnp.int32, (_HWP, _HWP), 1)
    valid = lane < _HW

    def feats(x):
        acc = fw[:, 0:1] * x[0:1, :] + fw[:, 1:2] * x[1:2, :] \
            + fw[:, 2:3] * x[2:3, :] + fb
        acc = jnp.maximum(acc, 0.0)
        acc = jnp.where(valid, acc, 0.0)
        return acc.astype(jnp.bfloat16)

    f1 = feats(x1)
    f2 = feats(x2)

    r = jax.lax.dot_general(f1, f2, (((0,), (0,)), ((), ())),
                            preferred_element_type=jnp.float32)
    r = jnp.maximum(r, 0.0)
    rr = r * r
    inv_col = jax.lax.rsqrt(jnp.sum(rr, axis=0, keepdims=True) + _EPS)
    inv_row = jax.lax.rsqrt(jnp.sum(rr, axis=1, keepdims=True) + _EPS)
    g1 = r * inv_col
    g2 = r * inv_row

    parts = [jnp.sum(g1 * w1_ref[f], axis=0, keepdims=True)
             for f in range(16)]
    parts += [jnp.sum(g2 * w2_ref[f], axis=0, keepdims=True)
              for f in range(16)]
    col = jnp.concatenate(parts, axis=0)
    ones_row = jnp.ones((1, col.shape[1]), jnp.float32)
    tot = jax.lax.dot_general(ones_row, col, (((1,), (1,)), ((), ())),
                              preferred_element_type=jnp.float32)
    bias = b_ref[...]
    o1_ref[0] = (tot[:, :16] + bias).astype(o1_ref.dtype)
    o2_ref[0] = (tot[:, 16:] + bias).astype(o2_ref.dtype)


def kernel(img1, img2, feat_w, feat_b, reg_w, reg_b):
    B = img1.shape[0]
    C = feat_w.shape[0]
    x1 = jnp.pad(img1.reshape(B, 3, _HW), ((0, 0), (0, 0), (0, _HWP - _HW)))
    x2 = jnp.pad(img2.reshape(B, 3, _HW), ((0, 0), (0, 0), (0, _HWP - _HW)))

    wq = reg_w.astype(jnp.bfloat16).astype(jnp.float32)
    r4 = wq.reshape(_H, _H, _HW, 16)
    w1 = jnp.transpose(r4, (3, 1, 0, 2)).reshape(16, _HW, _HW)
    w1 = jnp.pad(w1, ((0, 0), (0, _HWP - _HW), (0, _HWP - _HW)))
    w2 = jnp.transpose(w1, (0, 2, 1))

    fb2 = feat_b.reshape(C, 1)
    b2 = reg_b.reshape(1, 16).astype(jnp.float32)

    out1, out2 = pl.pallas_call(
        _fused_kernel,
        out_shape=(jax.ShapeDtypeStruct((B, 1, 16), img1.dtype),
                   jax.ShapeDtypeStruct((B, 1, 16), img1.dtype)),
        grid=(B,),
        in_specs=[pl.BlockSpec((1, 3, _HWP), lambda i: (i, 0, 0)),
                  pl.BlockSpec((1, 3, _HWP), lambda i: (i, 0, 0)),
                  pl.BlockSpec((C, 3), lambda i: (0, 0)),
                  pl.BlockSpec((C, 1), lambda i: (0, 0)),
                  pl.BlockSpec((16, _HWP, _HWP), lambda i: (0, 0, 0)),
                  pl.BlockSpec((16, _HWP, _HWP), lambda i: (0, 0, 0)),
                  pl.BlockSpec((1, 16), lambda i: (0, 0))],
        out_specs=(pl.BlockSpec((1, 1, 16), lambda i: (i, 0, 0)),
                   pl.BlockSpec((1, 1, 16), lambda i: (i, 0, 0))),
        compiler_params=pltpu.CompilerParams(
            dimension_semantics=("parallel",)),
    )(x1, x2, feat_w, fb2, w1, w2, b2)
    return out1.reshape(B, 16), out2.reshape(B, 16)


# speedup vs baseline: 3.8206x; 1.1438x over previous
import jax
import jax.numpy as jnp
from jax.experimental import pallas as pl
from jax.experimental.pallas import tpu as pltpu

_H = 15
_HW = _H * _H
_HWP = 256
_EPS = 1e-6
_P = 8


def _fused_kernel(xa_ref, fw_ref, w1_ref, b_ref, o1_ref, o2_ref):
    fw = fw_ref[...]
    bias = b_ref[...]
    ones_row = jnp.ones((1, _HWP), jnp.float32)

    for p in range(_P):
        x = xa_ref[p]
        f12 = jax.lax.dot_general(fw, x, (((1,), (0,)), ((), ())),
                                  preferred_element_type=jnp.float32)
        f12 = jnp.maximum(f12, 0.0).astype(jnp.bfloat16)
        f1 = f12[:, :_HWP]
        f2 = f12[:, _HWP:]

        r1 = jax.lax.dot_general(f1, f2, (((0,), (0,)), ((), ())),
                                 preferred_element_type=jnp.float32)
        r2 = jax.lax.dot_general(f2, f1, (((0,), (0,)), ((), ())),
                                 preferred_element_type=jnp.float32)
        r1 = jnp.maximum(r1, 0.0)
        r2 = jnp.maximum(r2, 0.0)
        q1 = r1 * r1
        q2 = r2 * r2
        s1 = jax.lax.dot_general(ones_row, q1, (((1,), (0,)), ((), ())),
                                 preferred_element_type=jnp.float32)
        s2 = jax.lax.dot_general(ones_row, q2, (((1,), (0,)), ((), ())),
                                 preferred_element_type=jnp.float32)
        g1 = r1 * jax.lax.rsqrt(s1 + _EPS)
        g2 = r2 * jax.lax.rsqrt(s2 + _EPS)

        parts1 = []
        parts2 = []
        for f in range(16):
            w = w1_ref[f]
            parts1.append(jnp.sum(g1 * w, axis=0, keepdims=True))
            parts2.append(jnp.sum(g2 * w, axis=0, keepdims=True))
        col = jnp.concatenate(parts1 + parts2, axis=0)
        tot = jax.lax.dot_general(ones_row, col, (((1,), (1,)), ((), ())),
                                  preferred_element_type=jnp.float32)
        o1_ref[p] = (tot[:, :16] + bias).astype(o1_ref.dtype)
        o2_ref[p] = (tot[:, 16:] + bias).astype(o2_ref.dtype)


def kernel(img1, img2, feat_w, feat_b, reg_w, reg_b):
    B = img1.shape[0]
    C = feat_w.shape[0]
    pad = _HWP - _HW
    x1 = jnp.pad(img1.reshape(B, 3, _HW), ((0, 0), (0, 0), (0, pad)))
    x2 = jnp.pad(img2.reshape(B, 3, _HW), ((0, 0), (0, 0), (0, pad)))
    ch = jnp.concatenate([x1, x2], axis=2)
    lane = jnp.arange(2 * _HWP) % _HWP
    ones_ch = jnp.broadcast_to((lane < _HW).astype(jnp.float32),
                               (B, 1, 2 * _HWP))
    xa = jnp.concatenate([ch, ones_ch], axis=1)

    fw = jnp.concatenate([feat_w, feat_b.reshape(C, 1)], axis=1)

    wq = reg_w.astype(jnp.bfloat16).astype(jnp.float32)
    r4 = wq.reshape(_H, _H, _HW, 16)
    w1 = jnp.transpose(r4, (3, 1, 0, 2)).reshape(16, _HW, _HW)
    w1 = jnp.pad(w1, ((0, 0), (0, pad), (0, pad)))

    b2 = reg_b.reshape(1, 16).astype(jnp.float32)

    out1, out2 = pl.pallas_call(
        _fused_kernel,
        out_shape=(jax.ShapeDtypeStruct((B, 1, 16), img1.dtype),
                   jax.ShapeDtypeStruct((B, 1, 16), img1.dtype)),
        grid=(B // _P,),
        in_specs=[pl.BlockSpec((_P, 4, 2 * _HWP), lambda i: (i, 0, 0)),
                  pl.BlockSpec((C, 4), lambda i: (0, 0)),
                  pl.BlockSpec((16, _HWP, _HWP), lambda i: (0, 0, 0)),
                  pl.BlockSpec((1, 16), lambda i: (0, 0))],
        out_specs=(pl.BlockSpec((_P, 1, 16), lambda i: (i, 0, 0)),
                   pl.BlockSpec((_P, 1, 16), lambda i: (i, 0, 0))),
        compiler_params=pltpu.CompilerParams(
            dimension_semantics=("parallel",)),
    )(xa, fw, w1, b2)
    return out1.reshape(B, 16), out2.reshape(B, 16)


# speedup vs baseline: 4.4230x; 1.1577x over previous
import jax
import jax.numpy as jnp
from jax.experimental import pallas as pl
from jax.experimental.pallas import tpu as pltpu

_H = 15
_HW = _H * _H
_HWP = 256
_EPS = 1e-6
_P = 8


def _fused_kernel(xa_ref, fw_ref, w1_ref, b_ref, o1_ref, o2_ref, g_ref):
    fw = fw_ref[...]
    bias = b_ref[...]
    ones_b = jnp.ones((1, _HWP), jnp.bfloat16)
    ones_f = jnp.ones((1, _HWP), jnp.float32)

    invs = [None] * (2 * _P)
    for p in range(_P):
        x = xa_ref[p]
        f12 = jax.lax.dot_general(fw, x, (((1,), (0,)), ((), ())),
                                  preferred_element_type=jnp.float32)
        f12 = jnp.maximum(f12, 0.0).astype(jnp.bfloat16)
        f1 = f12[:, :_HWP]
        f2 = f12[:, _HWP:]

        for d, (fa, fb) in enumerate(((f1, f2), (f2, f1))):
            r = jax.lax.dot_general(fa, fb, (((0,), (0,)), ((), ())),
                                    preferred_element_type=jnp.float32)
            rb = jnp.maximum(r, 0.0).astype(jnp.bfloat16)
            g_ref[2 * p + d] = rb
            q = rb * rb
            s = jax.lax.dot_general(ones_b, q, (((1,), (0,)), ((), ())),
                                    preferred_element_type=jnp.float32)
            invs[2 * p + d] = jax.lax.rsqrt(s + _EPS)

    parts = {}
    for p in range(_P):
        g1 = g_ref[2 * p]
        g2 = g_ref[2 * p + 1]
        for f in range(16):
            wf = w1_ref[f]
            parts[(p, 0, f)] = jax.lax.dot_general(
                ones_b, g1 * wf, (((1,), (0,)), ((), ())),
                preferred_element_type=jnp.float32)
            parts[(p, 1, f)] = jax.lax.dot_general(
                ones_b, g2 * wf, (((1,), (0,)), ((), ())),
                preferred_element_type=jnp.float32)

    for p in range(_P):
        smat = jnp.concatenate(
            [parts[(p, 0, f)] for f in range(16)]
            + [parts[(p, 1, f)] for f in range(16)], axis=0)
        scale = jnp.concatenate(
            [jnp.broadcast_to(invs[2 * p], (16, _HWP)),
             jnp.broadcast_to(invs[2 * p + 1], (16, _HWP))], axis=0)
        tmat = smat * scale
        tot = jax.lax.dot_general(ones_f, tmat, (((1,), (1,)), ((), ())),
                                  preferred_element_type=jnp.float32)
        o1_ref[p] = (tot[:, :16] + bias).astype(o1_ref.dtype)
        o2_ref[p] = (tot[:, 16:] + bias).astype(o2_ref.dtype)


def kernel(img1, img2, feat_w, feat_b, reg_w, reg_b):
    B = img1.shape[0]
    C = feat_w.shape[0]
    pad = _HWP - _HW
    x1 = jnp.pad(img1.reshape(B, 3, _HW), ((0, 0), (0, 0), (0, pad)))
    x2 = jnp.pad(img2.reshape(B, 3, _HW), ((0, 0), (0, 0), (0, pad)))
    ch = jnp.concatenate([x1, x2], axis=2)
    lane = jnp.arange(2 * _HWP) % _HWP
    ones_ch = jnp.broadcast_to((lane < _HW).astype(jnp.float32),
                               (B, 1, 2 * _HWP))
    xa = jnp.concatenate([ch, ones_ch], axis=1)

    fw = jnp.concatenate([feat_w, feat_b.reshape(C, 1)], axis=1)

    r4 = reg_w.reshape(_H, _H, _HW, 16)
    w1 = jnp.transpose(r4, (3, 1, 0, 2)).reshape(16, _HW, _HW)
    w1 = jnp.pad(w1, ((0, 0), (0, pad), (0, pad))).astype(jnp.bfloat16)

    b2 = reg_b.reshape(1, 16).astype(jnp.float32)

    out1, out2 = pl.pallas_call(
        _fused_kernel,
        out_shape=(jax.ShapeDtypeStruct((B, 1, 16), img1.dtype),
                   jax.ShapeDtypeStruct((B, 1, 16), img1.dtype)),
        grid=(B // _P,),
        in_specs=[pl.BlockSpec((_P, 4, 2 * _HWP), lambda i: (i, 0, 0)),
                  pl.BlockSpec((C, 4), lambda i: (0, 0)),
                  pl.BlockSpec((16, _HWP, _HWP), lambda i: (0, 0, 0)),
                  pl.BlockSpec((1, 16), lambda i: (0, 0))],
        out_specs=(pl.BlockSpec((_P, 1, 16), lambda i: (i, 0, 0)),
                   pl.BlockSpec((_P, 1, 16), lambda i: (i, 0, 0))),
        scratch_shapes=[pltpu.VMEM((2 * _P, _HWP, _HWP), jnp.bfloat16)],
        compiler_params=pltpu.CompilerParams(
            dimension_semantics=("parallel",)),
    )(xa, fw, w1, b2)
    return out1.reshape(B, 16), out2.reshape(B, 16)


# speedup vs baseline: 5.1126x; 1.1559x over previous
import jax
import jax.numpy as jnp
from jax.experimental import pallas as pl
from jax.experimental.pallas import tpu as pltpu

_H = 15
_HW = _H * _H
_HWP = 256
_EPS = 1e-6
_P = 16


def _fused_kernel(xa_ref, fw_ref, w1_ref, b_ref, o1_ref, o2_ref, g_ref,
                  f_ref):
    fw = fw_ref[...]
    bias = b_ref[...]
    ones_b = jnp.ones((1, _HWP), jnp.bfloat16)
    ones_f = jnp.ones((1, _HWP), jnp.float32)

    def feats_block(p):
        x = xa_ref[p]
        f12 = jax.lax.dot_general(fw, x, (((1,), (0,)), ((), ())),
                                  preferred_element_type=jnp.float32)
        f_ref[p] = jnp.maximum(f12, 0.0).astype(jnp.bfloat16)

    invs = [None] * (2 * _P)

    def corr_block(p):
        f1 = f_ref[p, :, :_HWP]
        f2 = f_ref[p, :, _HWP:]
        for d, (fa, fb) in enumerate(((f1, f2), (f2, f1))):
            r = jax.lax.dot_general(fa, fb, (((0,), (0,)), ((), ())),
                                    preferred_element_type=jnp.float32)
            rb = jnp.maximum(r, 0.0).astype(jnp.bfloat16)
            g_ref[2 * p + d] = rb
            q = rb * rb
            s = jax.lax.dot_general(ones_b, q, (((1,), (0,)), ((), ())),
                                    preferred_element_type=jnp.float32)
            invs[2 * p + d] = jax.lax.rsqrt(s + _EPS)

    parts = {}

    def products_block(p):
        g1 = g_ref[2 * p]
        g2 = g_ref[2 * p + 1]
        for f in range(16):
            wf = w1_ref[f]
            parts[(p, 0, f)] = jax.lax.dot_general(
                ones_b, g1 * wf, (((1,), (0,)), ((), ())),
                preferred_element_type=jnp.float32)
            parts[(p, 1, f)] = jax.lax.dot_general(
                ones_b, g2 * wf, (((1,), (0,)), ((), ())),
                preferred_element_type=jnp.float32)

    def finish_block(p):
        smat = jnp.concatenate(
            [parts[(p, 0, f)] for f in range(16)]
            + [parts[(p, 1, f)] for f in range(16)], axis=0)
        scale = jnp.concatenate(
            [jnp.broadcast_to(invs[2 * p], (16, _HWP)),
             jnp.broadcast_to(invs[2 * p + 1], (16, _HWP))], axis=0)
        tmat = smat * scale
        tot = jax.lax.dot_general(ones_f, tmat, (((1,), (1,)), ((), ())),
                                  preferred_element_type=jnp.float32)
        o1_ref[p] = (tot[:, :16] + bias).astype(o1_ref.dtype)
        o2_ref[p] = (tot[:, 16:] + bias).astype(o2_ref.dtype)

    for p in range(_P):
        feats_block(p)
    corr_block(0)
    for p in range(_P):
        if p + 1 < _P:
            corr_block(p + 1)
        products_block(p)
        if p >= 1:
            finish_block(p - 1)
    finish_block(_P - 1)


def kernel(img1, img2, feat_w, feat_b, reg_w, reg_b):
    B = img1.shape[0]
    C = feat_w.shape[0]
    pad = _HWP - _HW
    x1 = jnp.pad(img1.reshape(B, 3, _HW), ((0, 0), (0, 0), (0, pad)))
    x2 = jnp.pad(img2.reshape(B, 3, _HW), ((0, 0), (0, 0), (0, pad)))
    ch = jnp.concatenate([x1, x2], axis=2)
    lane = jnp.arange(2 * _HWP) % _HWP
    ones_ch = jnp.broadcast_to((lane < _HW).astype(jnp.float32),
                               (B, 1, 2 * _HWP))
    xa = jnp.concatenate([ch, ones_ch], axis=1)

    fw = jnp.concatenate([feat_w, feat_b.reshape(C, 1)], axis=1)

    r4 = reg_w.reshape(_H, _H, _HW, 16)
    w1 = jnp.transpose(r4, (3, 1, 0, 2)).reshape(16, _HW, _HW)
    w1 = jnp.pad(w1, ((0, 0), (0, pad), (0, pad))).astype(jnp.bfloat16)

    b2 = reg_b.reshape(1, 16).astype(jnp.float32)

    out1, out2 = pl.pallas_call(
        _fused_kernel,
        out_shape=(jax.ShapeDtypeStruct((B, 1, 16), img1.dtype),
                   jax.ShapeDtypeStruct((B, 1, 16), img1.dtype)),
        grid=(B // _P,),
        in_specs=[pl.BlockSpec((_P, 4, 2 * _HWP), lambda i: (i, 0, 0)),
                  pl.BlockSpec((C, 4), lambda i: (0, 0)),
                  pl.BlockSpec((16, _HWP, _HWP), lambda i: (0, 0, 0)),
                  pl.BlockSpec((1, 16), lambda i: (0, 0))],
        out_specs=(pl.BlockSpec((_P, 1, 16), lambda i: (i, 0, 0)),
                   pl.BlockSpec((_P, 1, 16), lambda i: (i, 0, 0))),
        scratch_shapes=[pltpu.VMEM((2 * _P, _HWP, _HWP), jnp.bfloat16),
                        pltpu.VMEM((_P, _HWP, 2 * _HWP), jnp.bfloat16)],
        compiler_params=pltpu.CompilerParams(
            dimension_semantics=("parallel",)),
    )(xa, fw, w1, b2)
    return out1.reshape(B, 16), out2.reshape(B, 16)


# speedup vs baseline: 5.2244x; 1.0219x over previous
import jax
import jax.numpy as jnp
from jax.experimental import pallas as pl
from jax.experimental.pallas import tpu as pltpu

_H = 15
_HW = _H * _H
_HWP = 256
_EPS = 1e-6
_P = 32


def _fused_kernel(xa_ref, fw_ref, w1_ref, b_ref, o1_ref, o2_ref, g_ref,
                  f_ref):
    fw = fw_ref[...]
    bias = b_ref[...]
    ones_b = jnp.ones((1, _HWP), jnp.bfloat16)
    ones_f = jnp.ones((1, _HWP), jnp.float32)

    def feats_block(p):
        x = xa_ref[p]
        f12 = jax.lax.dot_general(fw, x, (((1,), (0,)), ((), ())),
                                  preferred_element_type=jnp.float32)
        f_ref[p] = jnp.maximum(f12, 0.0).astype(jnp.bfloat16)

    invs = [None] * (2 * _P)

    def corr_block(p):
        f1 = f_ref[p, :, :_HWP]
        f2 = f_ref[p, :, _HWP:]
        for d, (fa, fb) in enumerate(((f1, f2), (f2, f1))):
            r = jax.lax.dot_general(fa, fb, (((0,), (0,)), ((), ())),
                                    preferred_element_type=jnp.float32)
            rb = jnp.maximum(r, 0.0).astype(jnp.bfloat16)
            g_ref[2 * p + d] = rb
            q = rb * rb
            s = jax.lax.dot_general(ones_b, q, (((1,), (0,)), ((), ())),
                                    preferred_element_type=jnp.float32)
            invs[2 * p + d] = jax.lax.rsqrt(s + _EPS)

    parts = {}

    def products_block(p):
        g1 = g_ref[2 * p]
        g2 = g_ref[2 * p + 1]
        for f in range(16):
            wf = w1_ref[f]
            parts[(p, 0, f)] = jax.lax.dot_general(
                ones_b, g1 * wf, (((1,), (0,)), ((), ())),
                preferred_element_type=jnp.float32)
            parts[(p, 1, f)] = jax.lax.dot_general(
                ones_b, g2 * wf, (((1,), (0,)), ((), ())),
                preferred_element_type=jnp.float32)

    def finish_block(p):
        smat = jnp.concatenate(
            [parts[(p, 0, f)] for f in range(16)]
            + [parts[(p, 1, f)] for f in range(16)], axis=0)
        scale = jnp.concatenate(
            [jnp.broadcast_to(invs[2 * p], (16, _HWP)),
             jnp.broadcast_to(invs[2 * p + 1], (16, _HWP))], axis=0)
        tmat = smat * scale
        tot = jax.lax.dot_general(ones_f, tmat, (((1,), (1,)), ((), ())),
                                  preferred_element_type=jnp.float32)
        o1_ref[p] = (tot[:, :16] + bias).astype(o1_ref.dtype)
        o2_ref[p] = (tot[:, 16:] + bias).astype(o2_ref.dtype)

    for p in range(_P):
        feats_block(p)
    corr_block(0)
    for p in range(_P):
        if p + 1 < _P:
            corr_block(p + 1)
        products_block(p)
        if p >= 1:
            finish_block(p - 1)
    finish_block(_P - 1)


def kernel(img1, img2, feat_w, feat_b, reg_w, reg_b):
    B = img1.shape[0]
    C = feat_w.shape[0]
    pad = _HWP - _HW
    x1 = jnp.pad(img1.reshape(B, 3, _HW), ((0, 0), (0, 0), (0, pad)))
    x2 = jnp.pad(img2.reshape(B, 3, _HW), ((0, 0), (0, 0), (0, pad)))
    ch = jnp.concatenate([x1, x2], axis=2)
    lane = jnp.arange(2 * _HWP) % _HWP
    ones_ch = jnp.broadcast_to((lane < _HW).astype(jnp.float32),
                               (B, 1, 2 * _HWP))
    xa = jnp.concatenate([ch, ones_ch], axis=1)

    fw = jnp.concatenate([feat_w, feat_b.reshape(C, 1)], axis=1)

    r4 = reg_w.reshape(_H, _H, _HW, 16)
    w1 = jnp.transpose(r4, (3, 1, 0, 2)).reshape(16, _HW, _HW)
    w1 = jnp.pad(w1, ((0, 0), (0, pad), (0, pad))).astype(jnp.bfloat16)

    b2 = reg_b.reshape(1, 16).astype(jnp.float32)

    out1, out2 = pl.pallas_call(
        _fused_kernel,
        out_shape=(jax.ShapeDtypeStruct((B, 1, 16), img1.dtype),
                   jax.ShapeDtypeStruct((B, 1, 16), img1.dtype)),
        grid=(B // _P,),
        in_specs=[pl.BlockSpec((_P, 4, 2 * _HWP), lambda i: (i, 0, 0)),
                  pl.BlockSpec((C, 4), lambda i: (0, 0)),
                  pl.BlockSpec((16, _HWP, _HWP), lambda i: (0, 0, 0)),
                  pl.BlockSpec((1, 16), lambda i: (0, 0))],
        out_specs=(pl.BlockSpec((_P, 1, 16), lambda i: (i, 0, 0)),
                   pl.BlockSpec((_P, 1, 16), lambda i: (i, 0, 0))),
        scratch_shapes=[pltpu.VMEM((2 * _P, _HWP, _HWP), jnp.bfloat16),
                        pltpu.VMEM((_P, _HWP, 2 * _HWP), jnp.bfloat16)],
        compiler_params=pltpu.CompilerParams(
            dimension_semantics=("parallel",)),
    )(xa, fw, w1, b2)
    return out1.reshape(B, 16), out2.reshape(B, 16)
